# baseline (speedup 1.0000x reference)
import functools

import jax
import jax.numpy as jnp
from jax import lax
from jax.experimental import pallas as pl
from jax.experimental.pallas import tpu as pltpu
from jax.experimental.pallas import tpu_sc as plsc

VOCAB = 100000
CTX = 200
DIM = 64
BATCH = 1024

NC = 2
NS = 16
NW = NC * NS
NTASK = CTX * (BATCH // 128)
TPW = NTASK // NW


NBUF = 4


def _sc_body(x_hbm, tok_hbm, pos_hbm, out_hbm,
             idxall, posb, gbufs, obufs, gsems, stsems):
  wid = lax.axis_index("s") * NC + lax.axis_index("c")
  base = wid * TPW

  pltpu.sync_copy(x_hbm.at[pl.ds(base, TPW)], idxall)
  pltpu.sync_copy(pos_hbm, posb)

  def fire(k, s):
    pltpu.async_copy(tok_hbm.at[idxall.at[k]], gbufs[s], gsems[s])

  def process(k, s, drain):
    gbuf, obuf = gbufs[s], obufs[s]
    tid = base + k
    t_hi = tid // 64
    rem = tid - t_hi * 64
    b_hi = rem // 8
    t_lo = rem - b_hi * 8
    t = t_hi * 8 + t_lo
    pltpu.make_async_copy(tok_hbm.at[pl.ds(0, 128)], gbuf, gsems[s]).wait()
    if drain:
      pltpu.make_async_copy(out_hbm.at[0, 0], obuf, stsems[s]).wait()

    pv = [posb[t, pl.ds(16 * j, 16)] for j in range(4)]
    cvec = [lax.iota(jnp.int32, 16) + (16 * j) for j in range(4)]

    @plsc.parallel_loop(0, 128, unroll=4)
    def _row(b):
      bvec = jnp.full((16,), b, jnp.int32)
      for j in range(4):
        v = gbuf[b, pl.ds(16 * j, 16)] + pv[j]
        plsc.store_scatter(obuf, [cvec[j], bvec], v)

    for ch in range(8):
      pltpu.async_copy(obuf.at[pl.ds(ch * 8, 8)],
                       out_hbm.at[t, ch, b_hi], stsems[s])

  for s in range(NBUF):
    fire(s, s)

  for s in range(NBUF):
    process(s, s, drain=False)
    fire(NBUF + s, s)

  @pl.loop(NBUF, 44, step=NBUF)
  def _quad(k):
    for s in range(NBUF):
      process(k + s, s, drain=True)
      fire(k + NBUF + s, s)

  process(44, 0, drain=True)
  fire(48, 0)
  process(45, 1, drain=True)
  fire(49, 1)
  process(46, 2, drain=True)
  process(47, 3, drain=True)
  process(48, 0, drain=True)
  process(49, 1, drain=True)

  for s in range(NBUF):
    pltpu.make_async_copy(out_hbm.at[0, 0], obufs[s], stsems[s]).wait()


@jax.jit
def _sc_embed(xrows, token_table, pos_table):
  mesh = plsc.VectorSubcoreMesh(core_axis_name="c", subcore_axis_name="s")
  return pl.kernel(
      _sc_body,
      out_type=jax.ShapeDtypeStruct((CTX, 8, 8, 8, 128), jnp.float32),
      mesh=mesh,
      scratch_types=[
          pltpu.VMEM((TPW, 128), jnp.int32),
          pltpu.VMEM((CTX, DIM), jnp.float32),
          [pltpu.VMEM((128, DIM), jnp.float32) for _ in range(NBUF)],
          [pltpu.VMEM((DIM, 128), jnp.float32) for _ in range(NBUF)],
          [pltpu.SemaphoreType.DMA for _ in range(NBUF)],
          [pltpu.SemaphoreType.DMA for _ in range(NBUF)],
      ],
      compiler_params=pltpu.CompilerParams(use_tc_tiling_on_sc=False, needs_layout_passes=False),
  )(xrows, token_table, pos_table)


def kernel(x, token_table, pos_table):
  xrows = (x.astype(jnp.int32)
            .reshape(8, 128, 25, 8)
            .transpose(2, 0, 3, 1)
            .reshape(NTASK, 128))
  out5 = _sc_embed(xrows, token_table, pos_table)
  return out5.transpose(2, 4, 0, 1, 3).reshape(BATCH, CTX, DIM)


# speedup vs baseline: 2.1127x; 2.1127x over previous
import functools

import jax
import jax.numpy as jnp
from jax import lax
from jax.experimental import pallas as pl
from jax.experimental.pallas import tpu as pltpu
from jax.experimental.pallas import tpu_sc as plsc

VOCAB = 100000
CTX = 200
DIM = 64
BATCH = 1024

NC = 2
NS = 16
NW = NC * NS
NTASK = CTX * (BATCH // 128)
TPW = NTASK // NW


NBUF = 4


def _sc_body(x_hbm, tok_hbm, pos_hbm, out_hbm,
             idxall, posb, gbufs, obufs, gsems, stsems):
  wid = lax.axis_index("s") * NC + lax.axis_index("c")
  base = wid * TPW

  pltpu.sync_copy(x_hbm.at[pl.ds(base, TPW)], idxall)
  pltpu.sync_copy(pos_hbm, posb)

  def fire(k, s):
    pltpu.async_copy(tok_hbm.at[idxall.at[k]], gbufs[s], gsems[s])

  def process(k, s, drain):
    gbuf, obuf = gbufs[s], obufs[s]
    tid = base + k
    t_hi = tid // 64
    rem = tid - t_hi * 64
    b_hi = rem // 8
    t_lo = rem - b_hi * 8
    t = t_hi * 8 + t_lo
    pltpu.make_async_copy(tok_hbm.at[pl.ds(0, 128)], gbuf, gsems[s]).wait()
    if drain:
      for ch in range(8):
        pltpu.make_async_copy(out_hbm.at[0, 0, 0],
                              obuf.at[pl.ds(ch * 8, 8), pl.ds(0, 128)],
                              stsems[s]).wait()

    pv = [posb[t, pl.ds(16 * j, 16)] for j in range(4)]
    cvec = [lax.iota(jnp.int32, 16) + (16 * j) for j in range(4)]

    @plsc.parallel_loop(0, 128, unroll=4)
    def _row(b):
      bvec = jnp.full((16,), b, jnp.int32)
      for j in range(4):
        v = gbuf[b, pl.ds(16 * j, 16)] + pv[j]
        plsc.store_scatter(obuf, [cvec[j], bvec], v)

    for ch in range(8):
      pltpu.async_copy(obuf.at[pl.ds(ch * 8, 8), pl.ds(0, 128)],
                       out_hbm.at[t, ch, b_hi], stsems[s])

  for s in range(NBUF):
    fire(s, s)

  for s in range(NBUF):
    process(s, s, drain=False)
    fire(NBUF + s, s)

  @pl.loop(NBUF, 44, step=NBUF)
  def _quad(k):
    for s in range(NBUF):
      process(k + s, s, drain=True)
      fire(k + NBUF + s, s)

  process(44, 0, drain=True)
  fire(48, 0)
  process(45, 1, drain=True)
  fire(49, 1)
  process(46, 2, drain=True)
  process(47, 3, drain=True)
  process(48, 0, drain=True)
  process(49, 1, drain=True)

  for s in range(NBUF):
    for ch in range(8):
      pltpu.make_async_copy(out_hbm.at[0, 0, 0],
                            obufs[s].at[pl.ds(ch * 8, 8), pl.ds(0, 128)],
                            stsems[s]).wait()


@jax.jit
def _sc_embed(xrows, token_table, pos_table):
  mesh = plsc.VectorSubcoreMesh(core_axis_name="c", subcore_axis_name="s")
  return pl.kernel(
      _sc_body,
      out_type=jax.ShapeDtypeStruct((CTX, 8, 8, 8, 128), jnp.float32),
      mesh=mesh,
      scratch_types=[
          pltpu.VMEM((TPW, 128), jnp.int32),
          pltpu.VMEM((CTX, DIM), jnp.float32),
          [pltpu.VMEM((128, DIM), jnp.float32) for _ in range(NBUF)],
          [pltpu.VMEM((DIM, 129), jnp.float32) for _ in range(NBUF)],
          [pltpu.SemaphoreType.DMA for _ in range(NBUF)],
          [pltpu.SemaphoreType.DMA for _ in range(NBUF)],
      ],
      compiler_params=pltpu.CompilerParams(use_tc_tiling_on_sc=False, needs_layout_passes=False),
  )(xrows, token_table, pos_table)


def kernel(x, token_table, pos_table):
  xrows = (x.astype(jnp.int32)
            .reshape(8, 128, 25, 8)
            .transpose(2, 0, 3, 1)
            .reshape(NTASK, 128))
  out5 = _sc_embed(xrows, token_table, pos_table)
  return out5.transpose(2, 4, 0, 1, 3).reshape(BATCH, CTX, DIM)


# speedup vs baseline: 2.1177x; 1.0024x over previous
import functools

import jax
import jax.numpy as jnp
from jax import lax
from jax.experimental import pallas as pl
from jax.experimental.pallas import tpu as pltpu
from jax.experimental.pallas import tpu_sc as plsc

VOCAB = 100000
CTX = 200
DIM = 64
BATCH = 1024

NC = 2
NS = 16
NW = NC * NS
NTASK = CTX * (BATCH // 128)
TPW = NTASK // NW


NBUF = 4


def _sc_body(x_hbm, tok_hbm, pos_hbm, out_hbm,
             idxall, posb, gbufs, obufs, gsems, stsems):
  wid = lax.axis_index("s") * NC + lax.axis_index("c")
  base = wid * TPW

  pltpu.sync_copy(x_hbm.at[pl.ds(base, TPW)], idxall)
  pltpu.sync_copy(pos_hbm, posb)

  def fire(k, s):
    pltpu.async_copy(tok_hbm.at[idxall.at[k]], gbufs[s], gsems[s])

  def process(k, s, drain):
    gbuf, obuf = gbufs[s], obufs[s]
    tid = base + k
    t_hi = tid // 64
    rem = tid - t_hi * 64
    b_hi = rem // 8
    t_lo = rem - b_hi * 8
    t = t_hi * 8 + t_lo
    pltpu.make_async_copy(tok_hbm.at[pl.ds(0, 128)], gbuf, gsems[s]).wait()
    if drain:
      for ch in range(8):
        pltpu.make_async_copy(out_hbm.at[0, 0, 0],
                              obuf.at[pl.ds(ch * 8, 8), pl.ds(0, 128)],
                              stsems[s]).wait()

    pv = [posb[pl.ds(t * DIM + 16 * j, 16)] for j in range(4)]
    cvec = [lax.iota(jnp.int32, 16) + (16 * j) for j in range(4)]

    @plsc.parallel_loop(0, 128, unroll=4)
    def _row(b):
      bvec = jnp.full((16,), b, jnp.int32)
      for j in range(4):
        v = gbuf[b, pl.ds(16 * j, 16)] + pv[j]
        plsc.store_scatter(obuf, [cvec[j], bvec], v)

    for ch in range(8):
      pltpu.async_copy(obuf.at[pl.ds(ch * 8, 8), pl.ds(0, 128)],
                       out_hbm.at[t, ch, b_hi], stsems[s])

  for s in range(NBUF):
    fire(s, s)

  for s in range(NBUF):
    process(s, s, drain=False)
    fire(NBUF + s, s)

  @pl.loop(NBUF, 44, step=NBUF)
  def _quad(k):
    for s in range(NBUF):
      process(k + s, s, drain=True)
      fire(k + NBUF + s, s)

  process(44, 0, drain=True)
  fire(48, 0)
  process(45, 1, drain=True)
  fire(49, 1)
  process(46, 2, drain=True)
  process(47, 3, drain=True)
  process(48, 0, drain=True)
  process(49, 1, drain=True)

  for s in range(NBUF):
    for ch in range(8):
      pltpu.make_async_copy(out_hbm.at[0, 0, 0],
                            obufs[s].at[pl.ds(ch * 8, 8), pl.ds(0, 128)],
                            stsems[s]).wait()


@jax.jit
def _sc_embed(xrows, token_table, pos_table):
  mesh = plsc.VectorSubcoreMesh(core_axis_name="c", subcore_axis_name="s")
  return pl.kernel(
      _sc_body,
      out_type=jax.ShapeDtypeStruct((CTX, 8, 8, 8, 128), jnp.float32),
      mesh=mesh,
      scratch_types=[
          pltpu.VMEM((TPW, 128), jnp.int32),
          pltpu.VMEM((CTX * DIM,), jnp.float32),
          [pltpu.VMEM((128, DIM), jnp.float32) for _ in range(NBUF)],
          [pltpu.VMEM((DIM, 129), jnp.float32) for _ in range(NBUF)],
          [pltpu.SemaphoreType.DMA for _ in range(NBUF)],
          [pltpu.SemaphoreType.DMA for _ in range(NBUF)],
      ],
      compiler_params=pltpu.CompilerParams(use_tc_tiling_on_sc=False, needs_layout_passes=False),
  )(xrows, token_table, pos_table)


def kernel(x, token_table, pos_table):
  xrows = (x.astype(jnp.int32)
            .reshape(8, 128, 25, 8)
            .transpose(2, 0, 3, 1)
            .reshape(NTASK, 128))
  out5 = _sc_embed(xrows, token_table, pos_table.reshape(-1))
  return out5.transpose(2, 4, 0, 1, 3).reshape(BATCH, CTX, DIM)
